# Initial kernel scaffold; baseline (speedup 1.0000x reference)
import functools

import jax
import jax.numpy as jnp
from jax import lax
from jax.experimental import pallas as pl
from jax.experimental.pallas import tpu as pltpu
from jax.experimental.pallas import tpu_sc as plsc

EPS_FACTOR = 2.0 * (1.0 + 0.1)

N_NODES = 10000
N_EDGES = 160000
D = 256
H = D // 2

NC = 2
NS = 16
EDGES_PER_TILE = N_EDGES // NS
CHUNK = 80
N_CHUNKS = EDGES_PER_TILE // CHUNK
ROWS_PER_TILE = N_NODES // NS
WB_CHUNK = 125
N_WB = ROWS_PER_TILE // WB_CHUNK


def _sc_segment_sum_body(xlo, xhi, src, dst, out, idx_s, idx_d, rows, stage, acc, sem):
  cid = lax.axis_index("c")
  sid = lax.axis_index("s")

  zero = jnp.zeros((16,), jnp.float32)

  def zero_row(r, _):
    for j in range(H // 16):
      stage[r, pl.ds(j * 16, 16)] = zero
    return 0

  lax.fori_loop(0, WB_CHUNK, zero_row, 0)
  rbase = sid * ROWS_PER_TILE
  for w in range(N_WB):
    pltpu.sync_copy(stage, acc.at[pl.ds(rbase + w * WB_CHUNK, WB_CHUNK)])
  plsc.subcore_barrier()

  ebase = sid * EDGES_PER_TILE

  def step(i, _):
    base = ebase + i * CHUNK
    pltpu.sync_copy(src.at[pl.ds(base, CHUNK)], idx_s)
    pltpu.sync_copy(dst.at[pl.ds(base, CHUNK)], idx_d)

    @pl.when(cid == 0)
    def _():
      pltpu.async_copy(xlo.at[idx_s], rows, sem).wait()

    @pl.when(cid == 1)
    def _():
      pltpu.async_copy(xhi.at[idx_s], rows, sem).wait()

    pltpu.sync_copy(rows, acc.at[idx_d], add=True)
    return 0

  lax.fori_loop(0, N_CHUNKS, step, 0)
  plsc.subcore_barrier()

  for w in range(N_WB):
    r = rbase + w * WB_CHUNK
    pltpu.sync_copy(acc.at[pl.ds(r, WB_CHUNK)], stage)
    pltpu.sync_copy(stage, out.at[cid, pl.ds(r, WB_CHUNK)])


@jax.jit
def _sc_segment_sum(xlo, xhi, src, dst):
  mesh = plsc.VectorSubcoreMesh(
      core_axis_name="c", subcore_axis_name="s", num_cores=NC, num_subcores=NS
  )
  return pl.kernel(
      _sc_segment_sum_body,
      out_type=jax.ShapeDtypeStruct((NC, N_NODES, H), jnp.float32),
      mesh=mesh,
      scratch_types=[
          pltpu.VMEM((CHUNK,), jnp.int32),
          pltpu.VMEM((CHUNK,), jnp.int32),
          pltpu.VMEM((CHUNK, H), jnp.float32),
          pltpu.VMEM((WB_CHUNK, H), jnp.float32),
          pltpu.VMEM_SHARED((N_NODES, H), jnp.float32),
          pltpu.SemaphoreType.DMA,
      ],
  )(xlo, xhi, src, dst)


def _dense_body(neigh_ref, x_ref, wt_ref, b_ref, o_ref):
  lo = neigh_ref[0] + EPS_FACTOR * x_ref[:, :H]
  hi = neigh_ref[1] + EPS_FACTOR * x_ref[:, H:]
  acc = jnp.dot(lo, wt_ref[:H, :], preferred_element_type=jnp.float32)
  acc = acc + jnp.dot(hi, wt_ref[H:, :], preferred_element_type=jnp.float32)
  o_ref[...] = jnp.maximum(acc + b_ref[...], 0.0)


@jax.jit
def _dense(neigh2, x, wt, b2):
  bn = 1000
  grid = (N_NODES // bn,)
  return pl.pallas_call(
      _dense_body,
      grid=grid,
      in_specs=[
          pl.BlockSpec((NC, bn, H), lambda i: (0, i, 0)),
          pl.BlockSpec((bn, D), lambda i: (i, 0)),
          pl.BlockSpec((D, D), lambda i: (0, 0)),
          pl.BlockSpec((1, D), lambda i: (0, 0)),
      ],
      out_specs=pl.BlockSpec((bn, D), lambda i: (i, 0)),
      out_shape=jax.ShapeDtypeStruct((N_NODES, D), jnp.float32),
  )(neigh2, x, wt, b2)


def kernel(x, edge_index, W, b):
  src = edge_index[0].astype(jnp.int32)
  dst = edge_index[1].astype(jnp.int32)
  xlo = x[:, :H]
  xhi = x[:, H:]
  neigh2 = _sc_segment_sum(xlo, xhi, src, dst)
  return _dense(neigh2, x, W.T, b.reshape(1, D))


# speedup vs baseline: 3.7759x; 3.7759x over previous
import functools

import jax
import jax.numpy as jnp
from jax import lax
from jax.experimental import pallas as pl
from jax.experimental.pallas import tpu as pltpu
from jax.experimental.pallas import tpu_sc as plsc

EPS_FACTOR = 2.0 * (1.0 + 0.1)

N_NODES = 10000
N_EDGES = 160000
D = 256
H = D // 2

NC = 2
NS = 16
EDGES_PER_TILE = N_EDGES // NS
CHUNK = 80
N_CHUNKS = EDGES_PER_TILE // CHUNK
WB = 80
NWB = N_NODES // WB
WB_PER_TILE = (NWB + NS - 1) // NS


def _sc_segment_sum_body(xlo, xhi, src, dst, out, idx_s, idx_d, rows, stage, acc, sem):
  cid = lax.axis_index("c")
  sid = lax.axis_index("s")

  zero = jnp.zeros((16,), jnp.float32)

  def zero_row(r, _):
    for j in range(H // 16):
      stage[r, pl.ds(j * 16, 16)] = zero
    return 0

  lax.fori_loop(0, WB, zero_row, 0)
  for k in range(WB_PER_TILE):
    c = sid + NS * k

    @pl.when(c < NWB)
    def _():
      pltpu.sync_copy(stage, acc.at[pl.ds(c * WB, WB)])

  plsc.subcore_barrier()

  ebase = sid * EDGES_PER_TILE

  def step(i, _):
    base = ebase + i * CHUNK
    pltpu.sync_copy(src.at[pl.ds(base, CHUNK)], idx_s)
    pltpu.sync_copy(dst.at[pl.ds(base, CHUNK)], idx_d)

    @pl.when(cid == 0)
    def _():
      pltpu.async_copy(xlo.at[idx_s], rows, sem).wait()

    @pl.when(cid == 1)
    def _():
      pltpu.async_copy(xhi.at[idx_s], rows, sem).wait()

    pltpu.sync_copy(rows, acc.at[idx_d], add=True)
    return 0

  lax.fori_loop(0, N_CHUNKS, step, 0)
  plsc.subcore_barrier()

  for k in range(WB_PER_TILE):
    c = sid + NS * k

    @pl.when(c < NWB)
    def _():
      pltpu.sync_copy(acc.at[pl.ds(c * WB, WB)], stage)
      pltpu.sync_copy(stage, out.at[cid, pl.ds(c * WB, WB)])


@jax.jit
def _sc_segment_sum(xlo, xhi, src, dst):
  mesh = plsc.VectorSubcoreMesh(
      core_axis_name="c", subcore_axis_name="s", num_cores=NC, num_subcores=NS
  )
  return pl.kernel(
      _sc_segment_sum_body,
      out_type=jax.ShapeDtypeStruct((NC, N_NODES, H), jnp.float32),
      mesh=mesh,
      scratch_types=[
          pltpu.VMEM((CHUNK,), jnp.int32),
          pltpu.VMEM((CHUNK,), jnp.int32),
          pltpu.VMEM((CHUNK, H), jnp.float32),
          pltpu.VMEM((WB, H), jnp.float32),
          pltpu.VMEM_SHARED((N_NODES, H), jnp.float32),
          pltpu.SemaphoreType.DMA,
      ],
  )(xlo, xhi, src, dst)


def _dense_body(neigh_ref, x_ref, wt_ref, b_ref, o_ref):
  lo = neigh_ref[0] + EPS_FACTOR * x_ref[:, :H]
  hi = neigh_ref[1] + EPS_FACTOR * x_ref[:, H:]
  acc = jnp.dot(lo, wt_ref[:H, :], preferred_element_type=jnp.float32)
  acc = acc + jnp.dot(hi, wt_ref[H:, :], preferred_element_type=jnp.float32)
  o_ref[...] = jnp.maximum(acc + b_ref[...], 0.0)


@jax.jit
def _dense(neigh2, x, wt, b2):
  bn = 1000
  grid = (N_NODES // bn,)
  return pl.pallas_call(
      _dense_body,
      grid=grid,
      in_specs=[
          pl.BlockSpec((NC, bn, H), lambda i: (0, i, 0)),
          pl.BlockSpec((bn, D), lambda i: (i, 0)),
          pl.BlockSpec((D, D), lambda i: (0, 0)),
          pl.BlockSpec((1, D), lambda i: (0, 0)),
      ],
      out_specs=pl.BlockSpec((bn, D), lambda i: (i, 0)),
      out_shape=jax.ShapeDtypeStruct((N_NODES, D), jnp.float32),
  )(neigh2, x, wt, b2)


def kernel(x, edge_index, W, b):
  src = edge_index[0].astype(jnp.int32)
  dst = edge_index[1].astype(jnp.int32)
  xlo = x[:, :H]
  xhi = x[:, H:]
  neigh2 = _sc_segment_sum(xlo, xhi, src, dst)
  return _dense(neigh2, x, W.T, b.reshape(1, D))


# speedup vs baseline: 7.2114x; 1.9098x over previous
import functools

import jax
import jax.numpy as jnp
from jax import lax
from jax.experimental import pallas as pl
from jax.experimental.pallas import tpu as pltpu
from jax.experimental.pallas import tpu_sc as plsc

EPS_FACTOR = 2.0 * (1.0 + 0.1)

N_NODES = 10000
N_EDGES = 160000
D = 256
H = D // 2

NC = 2
NS = 16
EDGES_PER_TILE = N_EDGES // NS
CHUNK = 80
N_CHUNKS = EDGES_PER_TILE // CHUNK
WB = 80
NWB = N_NODES // WB
WB_PER_TILE = (NWB + NS - 1) // NS


def _sc_segment_sum_body(
    x2, src3, dst, out, sbuf, db_a, db_b, rows_a, rows_b, acc, sem_a, sem_b
):
  cid = lax.axis_index("c")
  sid = lax.axis_index("s")
  xh = x2.at[cid]

  pltpu.sync_copy(src3.at[sid], sbuf)

  zero = jnp.zeros((16,), jnp.float32)

  def zero_row(r, _):
    for j in range(H // 16):
      rows_a[r, pl.ds(j * 16, 16)] = zero
    return 0

  lax.fori_loop(0, WB, zero_row, 0)
  for k in range(WB_PER_TILE):
    c = sid + NS * k

    @pl.when(c < NWB)
    def _():
      pltpu.sync_copy(rows_a, acc.at[pl.ds(c * WB, WB)])

  plsc.subcore_barrier()

  ebase = sid * EDGES_PER_TILE

  def gather(i, rows, sem):
    pltpu.async_copy(xh.at[sbuf.at[i]], rows, sem)

  def drain(rows, sem):
    pltpu.make_async_copy(xh.at[sbuf.at[0]], rows, sem).wait()

  def load_dst(i, db):
    pltpu.sync_copy(dst.at[pl.ds(ebase + i * CHUNK, CHUNK)], db)

  gather(0, rows_a, sem_a)

  def step(k, _):
    i = 2 * k
    gather(i + 1, rows_b, sem_b)
    load_dst(i, db_a)
    drain(rows_a, sem_a)
    pltpu.sync_copy(rows_a, acc.at[db_a], add=True)
    gather(i + 2, rows_a, sem_a)
    load_dst(i + 1, db_b)
    drain(rows_b, sem_b)
    pltpu.sync_copy(rows_b, acc.at[db_b], add=True)
    return 0

  lax.fori_loop(0, (N_CHUNKS - 1) // 2, step, 0)
  load_dst(N_CHUNKS - 1, db_a)
  drain(rows_a, sem_a)
  pltpu.sync_copy(rows_a, acc.at[db_a], add=True)
  plsc.subcore_barrier()

  for k in range(WB_PER_TILE):
    c = sid + NS * k

    @pl.when(c < NWB)
    def _():
      pltpu.sync_copy(acc.at[pl.ds(c * WB, WB)], rows_a)
      pltpu.sync_copy(rows_a, out.at[cid, pl.ds(c * WB, WB)])


@jax.jit
def _sc_segment_sum(x2, src3, dst):
  mesh = plsc.VectorSubcoreMesh(
      core_axis_name="c", subcore_axis_name="s", num_cores=NC, num_subcores=NS
  )
  return pl.kernel(
      _sc_segment_sum_body,
      out_type=jax.ShapeDtypeStruct((NC, N_NODES, H), jnp.float32),
      mesh=mesh,
      scratch_types=[
          pltpu.VMEM((N_CHUNKS, CHUNK), jnp.int32),
          pltpu.VMEM((CHUNK,), jnp.int32),
          pltpu.VMEM((CHUNK,), jnp.int32),
          pltpu.VMEM((CHUNK, H), jnp.float32),
          pltpu.VMEM((CHUNK, H), jnp.float32),
          pltpu.VMEM_SHARED((N_NODES, H), jnp.float32),
          pltpu.SemaphoreType.DMA,
          pltpu.SemaphoreType.DMA,
      ],
  )(x2, src3, dst)


def _dense_body(neigh_ref, x_ref, wt_ref, b_ref, o_ref):
  lo = neigh_ref[0] + EPS_FACTOR * x_ref[:, :H]
  hi = neigh_ref[1] + EPS_FACTOR * x_ref[:, H:]
  acc = jnp.dot(lo, wt_ref[:H, :], preferred_element_type=jnp.float32)
  acc = acc + jnp.dot(hi, wt_ref[H:, :], preferred_element_type=jnp.float32)
  o_ref[...] = jnp.maximum(acc + b_ref[...], 0.0)


@jax.jit
def _dense(neigh2, x, wt, b2):
  bn = 1000
  grid = (N_NODES // bn,)
  return pl.pallas_call(
      _dense_body,
      grid=grid,
      in_specs=[
          pl.BlockSpec((NC, bn, H), lambda i: (0, i, 0)),
          pl.BlockSpec((bn, D), lambda i: (i, 0)),
          pl.BlockSpec((D, D), lambda i: (0, 0)),
          pl.BlockSpec((1, D), lambda i: (0, 0)),
      ],
      out_specs=pl.BlockSpec((bn, D), lambda i: (i, 0)),
      out_shape=jax.ShapeDtypeStruct((N_NODES, D), jnp.float32),
  )(neigh2, x, wt, b2)


def kernel(x, edge_index, W, b):
  e32 = edge_index.astype(jnp.int32)
  src3 = e32[0].reshape(NS, N_CHUNKS, CHUNK)
  x2 = x.reshape(N_NODES, NC, H).transpose(1, 0, 2)
  neigh2 = _sc_segment_sum(x2, src3, e32[1])
  return _dense(neigh2, x, W.T, b.reshape(1, D))
